# baseline (speedup 1.0000x reference)
import math

import jax
import jax.numpy as jnp
import numpy as np
from jax import lax
from jax.experimental import pallas as pl
from jax.experimental.pallas import tpu as pltpu
from jax.experimental.pallas import tpu_sc as plsc

_VOCAB = 1000000
_MAX_LEN = 200
_EMB_DIM = 32
_BATCH = 4096

_NC = 2
_NS = 16
_NW = _NC * _NS
_LANES = 16

_TR = _MAX_LEN // 8
_BT = _BATCH // 128
_UNITS = _TR * _BT
_UNITS_PER_W = _UNITS // _NW
_UROWS = 8 * 128


def _pe_table() -> np.ndarray:
    dims = np.arange(0, _EMB_DIM, 2, dtype=np.float32)
    freq = np.exp(dims * (-math.log(10000.0) / _EMB_DIM))
    pos = np.arange(0, _MAX_LEN, dtype=np.float32)[:, None]
    pe = np.zeros((_MAX_LEN, _EMB_DIM), dtype=np.float32)
    pe[:, 0::2] = np.sin(pos * freq)
    pe[:, 1::2] = np.cos(pos * freq)
    return pe


_PE_CONST = _pe_table()


_TB = 2048
_TGRID = (_VOCAB + _TB - 1) // _TB


def _tc_transpose_body(wt_ref, o_ref):
    t3 = wt_ref[...].reshape(_EMB_DIM, _TB // 4, 4).transpose(1, 2, 0)
    o_ref[...] = t3.reshape(_TB // 4, 128)


def _sc_kernel(
    xi_hbm, w_hbm, pe_hbm, out_hbm,
    idx_a, idx_b, rows_a, rows_b, stage_v, pe_v, sem_a, sem_b, sem_o,
):
    wid = lax.axis_index("s") * _NC + lax.axis_index("c")
    base_u = wid * _UNITS_PER_W
    k16 = lax.iota(jnp.int32, _LANES)
    row_pat = k16 // 8
    sub_pat = k16 % 8

    pltpu.sync_copy(pe_hbm, pe_v)

    def _fetch(u, idx_v, rows_v, sem):
        pltpu.sync_copy(xi_hbm.at[pl.ds(u * _UROWS, _UROWS)], idx_v)
        return pltpu.async_copy(w_hbm.at[idx_v], rows_v, sem)

    def _process(u, rows_v, prev_out):
        tr = u // _BT
        bt = u % _BT
        if prev_out is not None:
            prev_out.wait()

        def _sl_body(sl, carry2):
            l = tr * 8 + sl
            pe0 = pe_v[l, pl.ds(0, _LANES)]
            pe1 = pe_v[l, pl.ds(_LANES, _LANES)]
            r_lo = row_pat + sl * 4
            r_hi = row_pat + (sl * 4 + 2)

            def _row_body(r, carry3):
                i = sl * 128 + r
                c = jnp.full((_LANES,), r, jnp.int32)
                v0 = rows_v[i, pl.ds(0, _LANES)] + pe0
                v1 = rows_v[i, pl.ds(_LANES, _LANES)] + pe1
                plsc.store_scatter(stage_v, [r_lo, sub_pat, c], v0)
                plsc.store_scatter(stage_v, [r_hi, sub_pat, c], v1)
                return carry3

            lax.fori_loop(0, 128, _row_body, carry2, unroll=8)
            return carry2

        lax.fori_loop(0, 8, _sl_body, 0)
        return pltpu.async_copy(
            stage_v.at[:, :, pl.ds(0, 128)],
            out_hbm.at[pl.ds(tr * 32, 32), bt, :, :],
            sem_o,
        )

    cp_a = _fetch(base_u, idx_a, rows_a, sem_a)
    out_cp = None
    for j in range(0, _UNITS_PER_W - 1, 2):
        cp_b = _fetch(base_u + j + 1, idx_b, rows_b, sem_b)
        cp_a.wait()
        out_cp = _process(base_u + j, rows_a, out_cp)
        cp_a = _fetch(base_u + j + 2, idx_a, rows_a, sem_a)
        cp_b.wait()
        out_cp = _process(base_u + j + 1, rows_b, out_cp)
    cp_a.wait()
    _process(base_u + _UNITS_PER_W - 1, rows_a, out_cp).wait()


@jax.jit
def _emb_lookup(x_img, w_t, pe):
    w4 = pl.pallas_call(
        _tc_transpose_body,
        grid=(_TGRID,),
        in_specs=[pl.BlockSpec((_EMB_DIM, _TB), lambda g: (0, g))],
        out_specs=pl.BlockSpec((_TB // 4, 128), lambda g: (g, 0)),
        out_shape=jax.ShapeDtypeStruct((_VOCAB * _EMB_DIM // 128, 128), jnp.float32),
    )(w_t)
    w = w4.reshape(_VOCAB, _EMB_DIM)
    mesh = plsc.VectorSubcoreMesh(core_axis_name="c", subcore_axis_name="s")
    f = pl.kernel(
        _sc_kernel,
        out_type=jax.ShapeDtypeStruct((_MAX_LEN * 4, _BT, 8, 128), jnp.float32),
        mesh=mesh,
        scratch_types=[
            pltpu.VMEM((_UROWS,), jnp.int32),
            pltpu.VMEM((_UROWS,), jnp.int32),
            pltpu.VMEM((_UROWS, _EMB_DIM), jnp.float32),
            pltpu.VMEM((_UROWS, _EMB_DIM), jnp.float32),
            pltpu.VMEM((32, 8, 129), jnp.float32),
            pltpu.VMEM((_MAX_LEN, _EMB_DIM), jnp.float32),
            pltpu.SemaphoreType.DMA,
            pltpu.SemaphoreType.DMA,
            pltpu.SemaphoreType.DMA,
        ],
        compiler_params=pltpu.CompilerParams(
            use_tc_tiling_on_sc=False, needs_layout_passes=False
        ),
    )
    return f(x_img, w, pe)


def kernel(x, W):
    x_img = (
        x.T.astype(jnp.int32)
        .reshape(_TR, 8, _BT, 128)
        .transpose(0, 2, 1, 3)
        .reshape(-1)
    )
    pe = jnp.asarray(_PE_CONST)
    out = _emb_lookup(x_img, W.T, pe)
    return (
        out.reshape(_MAX_LEN, 4, _BT, 8, 128)
        .transpose(2, 4, 0, 1, 3)
        .reshape(_BATCH, _MAX_LEN, _EMB_DIM)
    )


# speedup vs baseline: 3.8970x; 3.8970x over previous
import math

import jax
import jax.numpy as jnp
import numpy as np
from jax import lax
from jax.experimental import pallas as pl
from jax.experimental.pallas import tpu as pltpu
from jax.experimental.pallas import tpu_sc as plsc

_VOCAB = 1000000
_MAX_LEN = 200
_EMB_DIM = 32
_BATCH = 4096

_NC = 2
_NS = 16
_NW = _NC * _NS
_LANES = 16

_TR = _MAX_LEN // 8
_BT = _BATCH // 128
_UNITS = _TR * _BT
_UNITS_PER_W = _UNITS // _NW
_UROWS = 8 * 128


def _pe_table() -> np.ndarray:
    dims = np.arange(0, _EMB_DIM, 2, dtype=np.float32)
    freq = np.exp(dims * (-math.log(10000.0) / _EMB_DIM))
    pos = np.arange(0, _MAX_LEN, dtype=np.float32)[:, None]
    pe = np.zeros((_MAX_LEN, _EMB_DIM), dtype=np.float32)
    pe[:, 0::2] = np.sin(pos * freq)
    pe[:, 1::2] = np.cos(pos * freq)
    return pe


_PE_CONST = _pe_table()


_TB = 2048
_TGRID = (_VOCAB + _TB - 1) // _TB


def _tc_transpose_body(wt_ref, o_ref):
    t = wt_ref[...].T
    t4 = t.reshape(_TB // 4, 4, _EMB_DIM)
    parts = [t4[:, j, :] for j in range(4)]
    o_ref[...] = jnp.concatenate(parts, axis=1)


def _sc_kernel(
    xi_hbm, w_hbm, pe_hbm, out_hbm,
    idx_a, idx_b, rows_a, rows_b, stage_v, pe_v, sem_a, sem_b, sem_o,
):
    wid = lax.axis_index("s") * _NC + lax.axis_index("c")
    base_u = wid * _UNITS_PER_W
    k16 = lax.iota(jnp.int32, _LANES)
    row_pat = k16 // 8
    sub_pat = k16 % 8

    pltpu.sync_copy(pe_hbm, pe_v)

    def _fetch(u, idx_v, rows_v, sem):
        pltpu.sync_copy(xi_hbm.at[pl.ds(u * _UROWS, _UROWS)], idx_v)
        return pltpu.async_copy(w_hbm.at[idx_v], rows_v, sem)

    def _process(u, rows_v, prev_out):
        tr = u // _BT
        bt = u % _BT
        if prev_out is not None:
            prev_out.wait()

        def _sl_body(sl, carry2):
            l = tr * 8 + sl
            pe0 = pe_v[l, pl.ds(0, _LANES)]
            pe1 = pe_v[l, pl.ds(_LANES, _LANES)]
            r_lo = row_pat + sl * 4
            r_hi = row_pat + (sl * 4 + 2)

            def _row_body(r, carry3):
                i = sl * 128 + r
                c = jnp.full((_LANES,), r, jnp.int32)
                v0 = rows_v[i, pl.ds(0, _LANES)] + pe0
                v1 = rows_v[i, pl.ds(_LANES, _LANES)] + pe1
                plsc.store_scatter(stage_v, [r_lo, sub_pat, c], v0)
                plsc.store_scatter(stage_v, [r_hi, sub_pat, c], v1)
                return carry3

            lax.fori_loop(0, 128, _row_body, carry2, unroll=8)
            return carry2

        lax.fori_loop(0, 8, _sl_body, 0)
        return pltpu.async_copy(
            stage_v.at[:, :, pl.ds(0, 128)],
            out_hbm.at[pl.ds(tr * 32, 32), bt, :, :],
            sem_o,
        )

    cp_a = _fetch(base_u, idx_a, rows_a, sem_a)
    out_cp = None
    for j in range(0, _UNITS_PER_W - 1, 2):
        cp_b = _fetch(base_u + j + 1, idx_b, rows_b, sem_b)
        cp_a.wait()
        out_cp = _process(base_u + j, rows_a, out_cp)
        cp_a = _fetch(base_u + j + 2, idx_a, rows_a, sem_a)
        cp_b.wait()
        out_cp = _process(base_u + j + 1, rows_b, out_cp)
    cp_a.wait()
    _process(base_u + _UNITS_PER_W - 1, rows_a, out_cp).wait()


@jax.jit
def _emb_lookup(x_img, w_t, pe):
    w4 = pl.pallas_call(
        _tc_transpose_body,
        grid=(_TGRID,),
        in_specs=[pl.BlockSpec((_EMB_DIM, _TB), lambda g: (0, g))],
        out_specs=pl.BlockSpec((_TB // 4, 128), lambda g: (g, 0)),
        out_shape=jax.ShapeDtypeStruct((_VOCAB * _EMB_DIM // 128, 128), jnp.float32),
    )(w_t)
    w = w4.reshape(_VOCAB, _EMB_DIM)
    mesh = plsc.VectorSubcoreMesh(core_axis_name="c", subcore_axis_name="s")
    f = pl.kernel(
        _sc_kernel,
        out_type=jax.ShapeDtypeStruct((_MAX_LEN * 4, _BT, 8, 128), jnp.float32),
        mesh=mesh,
        scratch_types=[
            pltpu.VMEM((_UROWS,), jnp.int32),
            pltpu.VMEM((_UROWS,), jnp.int32),
            pltpu.VMEM((_UROWS, _EMB_DIM), jnp.float32),
            pltpu.VMEM((_UROWS, _EMB_DIM), jnp.float32),
            pltpu.VMEM((32, 8, 129), jnp.float32),
            pltpu.VMEM((_MAX_LEN, _EMB_DIM), jnp.float32),
            pltpu.SemaphoreType.DMA,
            pltpu.SemaphoreType.DMA,
            pltpu.SemaphoreType.DMA,
        ],
        compiler_params=pltpu.CompilerParams(
            use_tc_tiling_on_sc=False, needs_layout_passes=False
        ),
    )
    return f(x_img, w, pe)


def kernel(x, W):
    x_img = (
        x.T.astype(jnp.int32)
        .reshape(_TR, 8, _BT, 128)
        .transpose(0, 2, 1, 3)
        .reshape(-1)
    )
    pe = jnp.asarray(_PE_CONST)
    out = _emb_lookup(x_img, W.T, pe)
    return (
        out.reshape(_MAX_LEN, 4, _BT, 8, 128)
        .transpose(2, 4, 0, 1, 3)
        .reshape(_BATCH, _MAX_LEN, _EMB_DIM)
    )


# speedup vs baseline: 4.4496x; 1.1418x over previous
import math

import jax
import jax.numpy as jnp
import numpy as np
from jax import lax
from jax.experimental import pallas as pl
from jax.experimental.pallas import tpu as pltpu
from jax.experimental.pallas import tpu_sc as plsc

_VOCAB = 1000000
_MAX_LEN = 200
_EMB_DIM = 32
_BATCH = 4096

_NC = 2
_NS = 16
_NW = _NC * _NS
_LANES = 16

_TR = _MAX_LEN // 8
_BT = _BATCH // 128
_UNITS = _TR * _BT
_UNITS_PER_W = _UNITS // _NW
_UROWS = 8 * 128


def _pe_table() -> np.ndarray:
    dims = np.arange(0, _EMB_DIM, 2, dtype=np.float32)
    freq = np.exp(dims * (-math.log(10000.0) / _EMB_DIM))
    pos = np.arange(0, _MAX_LEN, dtype=np.float32)[:, None]
    pe = np.zeros((_MAX_LEN, _EMB_DIM), dtype=np.float32)
    pe[:, 0::2] = np.sin(pos * freq)
    pe[:, 1::2] = np.cos(pos * freq)
    return pe


_PE_CONST = _pe_table()


_TB = 8192
_TGRID = (_VOCAB + _TB - 1) // _TB


def _tc_transpose_body(wt_ref, o_ref):
    t = wt_ref[...].T
    t4 = t.reshape(_TB // 4, 4, _EMB_DIM)
    parts = [t4[:, j, :] for j in range(4)]
    o_ref[...] = jnp.concatenate(parts, axis=1)


def _sc_kernel(
    xi_hbm, w_hbm, pe_hbm, out_hbm,
    idx_a, idx_b, rows_a, rows_b, stage_v, pe_v, sem_a, sem_b, sem_o,
):
    wid = lax.axis_index("s") * _NC + lax.axis_index("c")
    base_u = wid * _UNITS_PER_W
    k16 = lax.iota(jnp.int32, _LANES)
    row_pat = k16 // 8
    sub_pat = k16 % 8

    pltpu.sync_copy(pe_hbm, pe_v)

    def _fetch(u, idx_v, rows_v, sem):
        pltpu.sync_copy(xi_hbm.at[pl.ds(u * _UROWS, _UROWS)], idx_v)
        return pltpu.async_copy(w_hbm.at[idx_v], rows_v, sem)

    def _process(u, rows_v, prev_out):
        tr = u // _BT
        bt = u % _BT
        if prev_out is not None:
            prev_out.wait()

        def _sl_body(sl, carry2):
            l = tr * 8 + sl
            pe0 = pe_v[l, pl.ds(0, _LANES)]
            pe1 = pe_v[l, pl.ds(_LANES, _LANES)]
            r_lo = row_pat + sl * 4
            r_hi = row_pat + (sl * 4 + 2)

            def _row_body(r, carry3):
                i = sl * 128 + r
                c = jnp.full((_LANES,), r, jnp.int32)
                v0 = rows_v[i, pl.ds(0, _LANES)] + pe0
                v1 = rows_v[i, pl.ds(_LANES, _LANES)] + pe1
                plsc.store_scatter(stage_v, [r_lo, sub_pat, c], v0)
                plsc.store_scatter(stage_v, [r_hi, sub_pat, c], v1)
                return carry3

            lax.fori_loop(0, 128, _row_body, carry2, unroll=8)
            return carry2

        lax.fori_loop(0, 8, _sl_body, 0)
        return pltpu.async_copy(
            stage_v.at[:, :, pl.ds(0, 128)],
            out_hbm.at[pl.ds(tr * 32, 32), bt, :, :],
            sem_o,
        )

    cp_a = _fetch(base_u, idx_a, rows_a, sem_a)
    out_cp = None
    for j in range(0, _UNITS_PER_W - 1, 2):
        cp_b = _fetch(base_u + j + 1, idx_b, rows_b, sem_b)
        cp_a.wait()
        out_cp = _process(base_u + j, rows_a, out_cp)
        cp_a = _fetch(base_u + j + 2, idx_a, rows_a, sem_a)
        cp_b.wait()
        out_cp = _process(base_u + j + 1, rows_b, out_cp)
    cp_a.wait()
    _process(base_u + _UNITS_PER_W - 1, rows_a, out_cp).wait()


@jax.jit
def _emb_lookup(x_img, w_t, pe):
    w4 = pl.pallas_call(
        _tc_transpose_body,
        grid=(_TGRID,),
        in_specs=[pl.BlockSpec((_EMB_DIM, _TB), lambda g: (0, g))],
        out_specs=pl.BlockSpec((_TB // 4, 128), lambda g: (g, 0)),
        out_shape=jax.ShapeDtypeStruct((_VOCAB * _EMB_DIM // 128, 128), jnp.float32),
    )(w_t)
    w = w4.reshape(_VOCAB, _EMB_DIM)
    mesh = plsc.VectorSubcoreMesh(core_axis_name="c", subcore_axis_name="s")
    f = pl.kernel(
        _sc_kernel,
        out_type=jax.ShapeDtypeStruct((_MAX_LEN * 4, _BT, 8, 128), jnp.float32),
        mesh=mesh,
        scratch_types=[
            pltpu.VMEM((_UROWS,), jnp.int32),
            pltpu.VMEM((_UROWS,), jnp.int32),
            pltpu.VMEM((_UROWS, _EMB_DIM), jnp.float32),
            pltpu.VMEM((_UROWS, _EMB_DIM), jnp.float32),
            pltpu.VMEM((32, 8, 129), jnp.float32),
            pltpu.VMEM((_MAX_LEN, _EMB_DIM), jnp.float32),
            pltpu.SemaphoreType.DMA,
            pltpu.SemaphoreType.DMA,
            pltpu.SemaphoreType.DMA,
        ],
        compiler_params=pltpu.CompilerParams(
            use_tc_tiling_on_sc=False, needs_layout_passes=False
        ),
    )
    return f(x_img, w, pe)


def kernel(x, W):
    x_img = (
        x.T.astype(jnp.int32)
        .reshape(_TR, 8, _BT, 128)
        .transpose(0, 2, 1, 3)
        .reshape(-1)
    )
    pe = jnp.asarray(_PE_CONST)
    out = _emb_lookup(x_img, W.T, pe)
    return (
        out.reshape(_MAX_LEN, 4, _BT, 8, 128)
        .transpose(2, 4, 0, 1, 3)
        .reshape(_BATCH, _MAX_LEN, _EMB_DIM)
    )


# speedup vs baseline: 4.5179x; 1.0153x over previous
import math

import jax
import jax.numpy as jnp
import numpy as np
from jax import lax
from jax.experimental import pallas as pl
from jax.experimental.pallas import tpu as pltpu
from jax.experimental.pallas import tpu_sc as plsc

_VOCAB = 1000000
_MAX_LEN = 200
_EMB_DIM = 32
_BATCH = 4096

_NC = 2
_NS = 16
_NW = _NC * _NS
_LANES = 16

_TR = _MAX_LEN // 8
_BT = _BATCH // 128
_UNITS = _TR * _BT
_UNITS_PER_W = _UNITS // _NW
_UROWS = 8 * 128


def _pe_table() -> np.ndarray:
    dims = np.arange(0, _EMB_DIM, 2, dtype=np.float32)
    freq = np.exp(dims * (-math.log(10000.0) / _EMB_DIM))
    pos = np.arange(0, _MAX_LEN, dtype=np.float32)[:, None]
    pe = np.zeros((_MAX_LEN, _EMB_DIM), dtype=np.float32)
    pe[:, 0::2] = np.sin(pos * freq)
    pe[:, 1::2] = np.cos(pos * freq)
    return pe


_PE_CONST = _pe_table()


_TB = 32768
_TGRID = (_VOCAB + _TB - 1) // _TB


def _tc_transpose_body(wt_ref, o_ref):
    t = wt_ref[...].T
    t4 = t.reshape(_TB // 4, 4, _EMB_DIM)
    parts = [t4[:, j, :] for j in range(4)]
    o_ref[...] = jnp.concatenate(parts, axis=1)


def _sc_kernel(
    xi_hbm, w_hbm, pe_hbm, out_hbm,
    idx_a, idx_b, rows_a, rows_b, stage_v, pe_v, sem_a, sem_b, sem_o,
):
    wid = lax.axis_index("s") * _NC + lax.axis_index("c")
    base_u = wid * _UNITS_PER_W
    k16 = lax.iota(jnp.int32, _LANES)
    row_pat = k16 // 8
    sub_pat = k16 % 8

    pltpu.sync_copy(pe_hbm, pe_v)

    def _fetch(u, idx_v, rows_v, sem):
        pltpu.sync_copy(xi_hbm.at[pl.ds(u * _UROWS, _UROWS)], idx_v)
        return pltpu.async_copy(w_hbm.at[idx_v], rows_v, sem)

    def _process(u, rows_v, prev_out):
        tr = u // _BT
        bt = u % _BT
        if prev_out is not None:
            prev_out.wait()

        def _sl_body(sl, carry2):
            l = tr * 8 + sl
            pe0 = pe_v[l, pl.ds(0, _LANES)]
            pe1 = pe_v[l, pl.ds(_LANES, _LANES)]
            r_lo = row_pat + sl * 4
            r_hi = row_pat + (sl * 4 + 2)

            def _row_body(r, carry3):
                i = sl * 128 + r
                c = jnp.full((_LANES,), r, jnp.int32)
                v0 = rows_v[i, pl.ds(0, _LANES)] + pe0
                v1 = rows_v[i, pl.ds(_LANES, _LANES)] + pe1
                plsc.store_scatter(stage_v, [r_lo, sub_pat, c], v0)
                plsc.store_scatter(stage_v, [r_hi, sub_pat, c], v1)
                return carry3

            lax.fori_loop(0, 128, _row_body, carry2, unroll=8)
            return carry2

        lax.fori_loop(0, 8, _sl_body, 0)
        return pltpu.async_copy(
            stage_v.at[:, :, pl.ds(0, 128)],
            out_hbm.at[pl.ds(tr * 32, 32), bt, :, :],
            sem_o,
        )

    cp_a = _fetch(base_u, idx_a, rows_a, sem_a)
    out_cp = None
    for j in range(0, _UNITS_PER_W - 1, 2):
        cp_b = _fetch(base_u + j + 1, idx_b, rows_b, sem_b)
        cp_a.wait()
        out_cp = _process(base_u + j, rows_a, out_cp)
        cp_a = _fetch(base_u + j + 2, idx_a, rows_a, sem_a)
        cp_b.wait()
        out_cp = _process(base_u + j + 1, rows_b, out_cp)
    cp_a.wait()
    _process(base_u + _UNITS_PER_W - 1, rows_a, out_cp).wait()


@jax.jit
def _emb_lookup(x_img, w_t, pe):
    w4 = pl.pallas_call(
        _tc_transpose_body,
        grid=(_TGRID,),
        in_specs=[pl.BlockSpec((_EMB_DIM, _TB), lambda g: (0, g))],
        out_specs=pl.BlockSpec((_TB // 4, 128), lambda g: (g, 0)),
        out_shape=jax.ShapeDtypeStruct((_VOCAB * _EMB_DIM // 128, 128), jnp.float32),
    )(w_t)
    w = w4.reshape(_VOCAB, _EMB_DIM)
    mesh = plsc.VectorSubcoreMesh(core_axis_name="c", subcore_axis_name="s")
    f = pl.kernel(
        _sc_kernel,
        out_type=jax.ShapeDtypeStruct((_MAX_LEN * 4, _BT, 8, 128), jnp.float32),
        mesh=mesh,
        scratch_types=[
            pltpu.VMEM((_UROWS,), jnp.int32),
            pltpu.VMEM((_UROWS,), jnp.int32),
            pltpu.VMEM((_UROWS, _EMB_DIM), jnp.float32),
            pltpu.VMEM((_UROWS, _EMB_DIM), jnp.float32),
            pltpu.VMEM((32, 8, 129), jnp.float32),
            pltpu.VMEM((_MAX_LEN, _EMB_DIM), jnp.float32),
            pltpu.SemaphoreType.DMA,
            pltpu.SemaphoreType.DMA,
            pltpu.SemaphoreType.DMA,
        ],
        compiler_params=pltpu.CompilerParams(
            use_tc_tiling_on_sc=False, needs_layout_passes=False
        ),
    )
    return f(x_img, w, pe)


def kernel(x, W):
    x_img = (
        x.T.astype(jnp.int32)
        .reshape(_TR, 8, _BT, 128)
        .transpose(0, 2, 1, 3)
        .reshape(-1)
    )
    pe = jnp.asarray(_PE_CONST)
    out = _emb_lookup(x_img, W.T, pe)
    return (
        out.reshape(_MAX_LEN, 4, _BT, 8, 128)
        .transpose(2, 4, 0, 1, 3)
        .reshape(_BATCH, _MAX_LEN, _EMB_DIM)
    )
